# baseline (speedup 1.0000x reference)
import functools

import jax
import jax.numpy as jnp
from jax import lax
from jax.experimental import pallas as pl
from jax.experimental.pallas import tpu as pltpu
from jax.experimental.pallas import tpu_sc as plsc

_NC = 2
_NS = 16
_NW = _NC * _NS
_W = 128


def kernel(word_input, embed_weight):
    B, S = word_input.shape
    V, D = embed_weight.shape
    N = B * S
    idx = word_input.reshape(N).astype(jnp.int32)
    table2 = embed_weight.reshape(V // 2, 2 * D)

    n_per_w = N // _NW
    chunks = n_per_w // _W
    mesh = plsc.VectorSubcoreMesh(core_axis_name="c", subcore_axis_name="s")

    @functools.partial(
        pl.kernel,
        out_type=jax.ShapeDtypeStruct((N, 2 * D), embed_weight.dtype),
        mesh=mesh,
        scratch_types=[
            pltpu.VMEM((n_per_w,), jnp.int32),
            pltpu.VMEM((2, _W, 2 * D), jnp.float32),
            pltpu.SemaphoreType.DMA,
        ],
    )
    def gather_kernel(table_hbm, idx_hbm, out_hbm, idx_v, rows_v, sem):
        wid = lax.axis_index("s") * _NC + lax.axis_index("c")
        base = wid * n_per_w
        pltpu.sync_copy(idx_hbm.at[pl.ds(base, n_per_w)], idx_v)

        @pl.loop(0, chunks)
        def _(c):
            win = idx_v.at[pl.ds(c * _W, _W)]
            pltpu.async_copy(table_hbm.at[win], rows_v.at[0], sem).wait()
            pltpu.sync_copy(rows_v.at[0], out_hbm.at[pl.ds(base + c * _W, _W)])

    fused = gather_kernel(table2, idx >> 1)
    out = jnp.where((idx & 1)[:, None] == 1, fused[:, D:], fused[:, :D])
    return out.reshape(B, S, D)


# speedup vs baseline: 1.0356x; 1.0356x over previous
import functools

import jax
import jax.numpy as jnp
from jax import lax
from jax.experimental import pallas as pl
from jax.experimental.pallas import tpu as pltpu
from jax.experimental.pallas import tpu_sc as plsc

_NC = 2
_NS = 16
_NW = _NC * _NS
_W = 128


def kernel(word_input, embed_weight):
    B, S = word_input.shape
    V, D = embed_weight.shape
    N = B * S
    idx = word_input.reshape(N).astype(jnp.int32)
    table2 = embed_weight.reshape(V // 2, 2 * D)

    n_per_w = N // _NW
    chunks = n_per_w // _W
    mesh = plsc.VectorSubcoreMesh(core_axis_name="c", subcore_axis_name="s")

    @functools.partial(
        pl.kernel,
        out_type=jax.ShapeDtypeStruct((N, 2 * D), embed_weight.dtype),
        mesh=mesh,
        scratch_types=[
            pltpu.VMEM((n_per_w,), jnp.int32),
            pltpu.VMEM((2, _W, 2 * D), jnp.float32),
            pltpu.SemaphoreType.DMA,
            pltpu.SemaphoreType.DMA,
        ],
    )
    def gather_kernel(table_hbm, idx_hbm, out_hbm, idx_v, rows_v, sem0, sem1):
        wid = lax.axis_index("s") * _NC + lax.axis_index("c")
        base = wid * n_per_w
        pltpu.sync_copy(idx_hbm.at[pl.ds(base, n_per_w)], idx_v)

        def fire(c, buf, sem):
            win = idx_v.at[pl.ds(c * _W, _W)]
            return pltpu.async_copy(table_hbm.at[win], rows_v.at[buf], sem)

        def drain(c, buf, sem):
            win = idx_v.at[pl.ds(c * _W, _W)]
            pltpu.make_async_copy(table_hbm.at[win], rows_v.at[buf], sem).wait()
            pltpu.sync_copy(rows_v.at[buf], out_hbm.at[pl.ds(base + c * _W, _W)])

        fire(0, 0, sem0)

        @pl.loop(0, chunks // 2 - 1)
        def _(h):
            c = 2 * h
            fire(c + 1, 1, sem1)
            drain(c, 0, sem0)
            fire(c + 2, 0, sem0)
            drain(c + 1, 1, sem1)

        fire(chunks - 1, 1, sem1)
        drain(chunks - 2, 0, sem0)
        drain(chunks - 1, 1, sem1)

    fused = gather_kernel(table2, idx >> 1)
    out = jnp.where((idx & 1)[:, None] == 1, fused[:, D:], fused[:, :D])
    return out.reshape(B, S, D)


# speedup vs baseline: 1.0390x; 1.0034x over previous
import dataclasses
import functools

import jax
import jax.numpy as jnp
from jax import lax
from jax.experimental import pallas as pl
from jax.experimental.pallas import tpu as pltpu
from jax.experimental.pallas import tpu_sc as plsc

_NC = 2
_NS = 16
_NW = _NC * _NS
_W = 128
_L = 16


def kernel(word_input, embed_weight):
    B, S = word_input.shape
    V, D = embed_weight.shape
    N = B * S
    idx = word_input.T.reshape(N)
    table2 = embed_weight.reshape(V // 2, 2 * D)

    n_per_w = N // _NW
    chunks = n_per_w // _W
    mesh = plsc.VectorSubcoreMesh(core_axis_name="c", subcore_axis_name="s")

    cp = pltpu.CompilerParams()
    if "needs_layout_passes" in pltpu.CompilerParams.__dataclass_fields__:
        cp = dataclasses.replace(cp, needs_layout_passes=False)

    @functools.partial(
        pl.kernel,
        out_type=jax.ShapeDtypeStruct((S, D, B), embed_weight.dtype),
        mesh=mesh,
        compiler_params=cp,
        scratch_types=[
            pltpu.VMEM((n_per_w,), jnp.int32),
            pltpu.VMEM((n_per_w,), jnp.int32),
            pltpu.VMEM((2, _W, 2 * D), jnp.float32),
            pltpu.VMEM((2, D, _W), jnp.float32),
            pltpu.SemaphoreType.DMA,
            pltpu.SemaphoreType.DMA,
            pltpu.SemaphoreType.DMA,
            pltpu.SemaphoreType.DMA,
        ],
    )
    def gather_kernel(table_hbm, idx_hbm, out_hbm, idx_v, fidx_v, rows_v,
                      wout_v, gsem0, gsem1, wsem0, wsem1):
        wid = lax.axis_index("s") * _NC + lax.axis_index("c")
        base = wid * n_per_w
        pltpu.sync_copy(idx_hbm.at[pl.ds(base, n_per_w)], idx_v)

        @pl.loop(0, n_per_w // _L)
        def _(k):
            v = idx_v.at[pl.ds(k * _L, _L)][...]
            fidx_v.at[pl.ds(k * _L, _L)][...] = lax.shift_right_logical(v, 1)

        def fire(c, buf, sem):
            win = fidx_v.at[pl.ds(c * _W, _W)]
            pltpu.async_copy(table_hbm.at[win], rows_v.at[buf], sem)

        iota = lax.iota(jnp.int32, _L)

        def transform(c, buf):
            rows = rows_v.at[buf]
            wout = wout_v.at[buf]
            rowsel = []
            colbase = []
            for g in range(_W // _L):
                rowsel.append(g * _L + iota)
                pv = idx_v.at[pl.ds(c * _W + g * _L, _L)][...]
                colbase.append((pv & 1) * D)

            @pl.loop(0, D)
            def _(d):
                for g in range(_W // _L):
                    val = plsc.load_gather(rows, [rowsel[g], colbase[g] + d])
                    wout.at[d, pl.ds(g * _L, _L)][...] = val

        def drain(c, buf, gsem, wsem):
            win = fidx_v.at[pl.ds(c * _W, _W)]
            pltpu.make_async_copy(table_hbm.at[win], rows_v.at[buf], gsem).wait()
            transform(c, buf)
            m0 = base + c * _W
            s = m0 // B
            b0 = m0 % B
            pltpu.async_copy(wout_v.at[buf], out_hbm.at[s, :, pl.ds(b0, _W)], wsem)

        def wait_write(c, buf, wsem):
            m0 = base + c * _W
            s = m0 // B
            b0 = m0 % B
            pltpu.make_async_copy(
                wout_v.at[buf], out_hbm.at[s, :, pl.ds(b0, _W)], wsem).wait()

        fire(0, 0, gsem0)
        fire(1, 1, gsem1)
        drain(0, 0, gsem0, wsem0)
        drain(1, 1, gsem1, wsem1)

        @pl.loop(1, chunks // 2)
        def _(h):
            c = 2 * h
            fire(c, 0, gsem0)
            fire(c + 1, 1, gsem1)
            wait_write(c - 2, 0, wsem0)
            drain(c, 0, gsem0, wsem0)
            wait_write(c - 1, 1, wsem1)
            drain(c + 1, 1, gsem1, wsem1)

        wait_write(chunks - 2, 0, wsem0)
        wait_write(chunks - 1, 1, wsem1)

    out3d = gather_kernel(table2, idx)
    return jnp.transpose(out3d, (2, 0, 1))


# speedup vs baseline: 2.2338x; 2.1498x over previous
import functools

import jax
import jax.numpy as jnp
from jax import lax
from jax.experimental import pallas as pl
from jax.experimental.pallas import tpu as pltpu
from jax.experimental.pallas import tpu_sc as plsc

_NC = 2
_NS = 16
_NW = _NC * _NS
_W = 128
_K = 8192


def _pad_block(a_ref, o_ref):
    t = jnp.swapaxes(a_ref[...], 0, 1)
    o_ref[...] = jnp.concatenate([t, t], axis=1)


def kernel(word_input, embed_weight):
    B, S = word_input.shape
    V, D = embed_weight.shape
    N = B * S
    idx = word_input.T.reshape(N)
    table_t = embed_weight.T

    table_p = pl.pallas_call(
        _pad_block,
        grid=(pl.cdiv(V, _K),),
        in_specs=[pl.BlockSpec((D, _K), lambda i: (0, i))],
        out_specs=pl.BlockSpec((_K, 2 * D), lambda i: (i, 0)),
        out_shape=jax.ShapeDtypeStruct((V, 2 * D), embed_weight.dtype),
    )(table_t)

    n_per_w = N // _NW
    chunks = n_per_w // _W
    mesh = plsc.VectorSubcoreMesh(core_axis_name="c", subcore_axis_name="s")

    @functools.partial(
        pl.kernel,
        out_type=jax.ShapeDtypeStruct((N, 2 * D), embed_weight.dtype),
        mesh=mesh,
        scratch_types=[
            pltpu.VMEM((n_per_w,), jnp.int32),
            pltpu.VMEM((2, _W, 2 * D), jnp.float32),
            pltpu.SemaphoreType.DMA,
            pltpu.SemaphoreType.DMA,
        ],
    )
    def gather_kernel(table_hbm, idx_hbm, out_hbm, idx_v, rows_v, sem0, sem1):
        wid = lax.axis_index("s") * _NC + lax.axis_index("c")
        base = wid * n_per_w
        pltpu.sync_copy(idx_hbm.at[pl.ds(base, n_per_w)], idx_v)

        def fire(c, buf, sem):
            win = idx_v.at[pl.ds(c * _W, _W)]
            pltpu.async_copy(table_hbm.at[win], rows_v.at[buf], sem)

        def drain(c, buf, sem):
            win = idx_v.at[pl.ds(c * _W, _W)]
            pltpu.make_async_copy(table_hbm.at[win], rows_v.at[buf], sem).wait()
            pltpu.sync_copy(rows_v.at[buf], out_hbm.at[pl.ds(base + c * _W, _W)])

        fire(0, 0, sem0)

        @pl.loop(0, chunks // 2 - 1)
        def _(h):
            c = 2 * h
            fire(c + 1, 1, sem1)
            drain(c, 0, sem0)
            fire(c + 2, 0, sem0)
            drain(c + 1, 1, sem1)

        fire(chunks - 1, 1, sem1)
        drain(chunks - 2, 0, sem0)
        drain(chunks - 1, 1, sem1)

    fused = gather_kernel(table_p, idx)
    out = fused.reshape(S, B, 2 * D)[:, :, :D]
    return jnp.transpose(out, (1, 0, 2))
